# baseline (speedup 1.0000x reference)
import jax
import jax.numpy as jnp
import numpy as np
from jax import lax
from jax.experimental import pallas as pl
from jax.experimental.pallas import tpu as pltpu
from jax.experimental.pallas import tpu_sc as plsc

_N = 2 * 1 * 128 * 128 * 128
_H = 2048
_W = 2048
_BH = 256

_NB = 65536
_SHIFT = 15
_NC = 2
_NS = 16
_NTILES = _NC * _NS
_PER_TILE = _N // _NTILES
_CHUNK = 8192
_NCHUNK = _PER_TILE // _CHUNK
_L = 16
_U = 8


_SHAPE = (2, 1, 128, 128, 128)
_BD = 32


def _bce_keys_body(l_ref, t_ref, key_ref, possum_ref):
    l = l_ref[...]
    t = t_ref[...]
    loss = jnp.maximum(l, 0.0) - l * t + jnp.log1p(jnp.exp(-jnp.abs(l)))
    neg = t <= 0.5
    digit_neg = lax.shift_right_logical(
        lax.bitcast_convert_type(loss, jnp.int32), _SHIFT)
    lane = lax.broadcasted_iota(jnp.int32, l.shape, dimension=4)
    digit_pos = jnp.int32(65280) + jnp.bitwise_and(lane, jnp.int32(127))
    key_ref[...] = jnp.where(neg, digit_neg, digit_pos)
    ps = jnp.sum(jnp.where(neg, 0.0, loss))

    @pl.when((pl.program_id(0) == 0) & (pl.program_id(1) == 0))
    def _():
        possum_ref[...] = jnp.zeros_like(possum_ref)

    possum_ref[...] += ps.reshape(1, 1)


def _make_bce(d0):
    return pl.pallas_call(
        _bce_keys_body,
        grid=(d0, 128 // _BD),
        in_specs=[
            pl.BlockSpec((1, 1, _BD, 128, 128), lambda i, j: (i, 0, j, 0, 0)),
            pl.BlockSpec((1, 1, _BD, 128, 128), lambda i, j: (i, 0, j, 0, 0)),
        ],
        out_specs=[
            pl.BlockSpec((1, 1, _BD, 128, 128), lambda i, j: (i, 0, j, 0, 0)),
            pl.BlockSpec((1, 1), lambda i, j: (0, 0)),
        ],
        out_shape=[
            jax.ShapeDtypeStruct((d0, 1, 128, 128, 128), jnp.int32),
            jax.ShapeDtypeStruct((1, 1), jnp.float32),
        ],
        compiler_params=pltpu.CompilerParams(
            dimension_semantics=("arbitrary", "arbitrary"),
        ),
    )


_bce_keys_half = _make_bce(1)


def _make_hist(n_elems):
    per_tile = n_elems // _NTILES
    nchunk = per_tile // _CHUNK

    def _hist_body(keys_hbm, cnt_out, chunk0, chunk1, cnt_v, sem0, sem1):
        wid = lax.axis_index("s") * _NC + lax.axis_index("c")
        zeros = jnp.zeros((_L,), jnp.float32)
        ones = jnp.ones((_L,), jnp.float32)

        def zero_body(i, carry):
            b = i * (_L * _U)
            for u in range(_U):
                cnt_v[pl.ds(b + u * _L, _L)] = zeros
            return carry

        lax.fori_loop(0, _NB // (_L * _U), zero_body, 0)

        base = wid * per_tile
        bufs = (chunk0, chunk1)
        sems = (sem0, sem1)

        def process(chunk_v):
            def vec_body(g, carry):
                b = g * (_L * _U)
                digits = [chunk_v[pl.ds(b + u * _L, _L)] for u in range(_U)]
                for digit in digits:
                    plsc.addupdate_scatter(cnt_v, [digit], ones)
                return carry

            lax.fori_loop(0, _CHUNK // (_L * _U), vec_body, 0)

        handles = [None] * nchunk
        handles[0] = pltpu.async_copy(
            keys_hbm.at[pl.ds(base, _CHUNK)], bufs[0], sems[0])
        for c in range(nchunk):
            if c + 1 < nchunk:
                handles[c + 1] = pltpu.async_copy(
                    keys_hbm.at[pl.ds(base + (c + 1) * _CHUNK, _CHUNK)],
                    bufs[(c + 1) % 2], sems[(c + 1) % 2])
            handles[c].wait()
            process(bufs[c % 2])

        pltpu.sync_copy(cnt_v, cnt_out.at[wid])

    return pl.kernel(
        _hist_body,
        out_type=jax.ShapeDtypeStruct((_NTILES, _NB), jnp.float32),
        mesh=plsc.VectorSubcoreMesh(core_axis_name="c", subcore_axis_name="s"),
        scratch_types=[
            pltpu.VMEM((_CHUNK,), jnp.int32),
            pltpu.VMEM((_CHUNK,), jnp.int32),
            pltpu.VMEM((_NB,), jnp.float32),
            pltpu.SemaphoreType.DMA,
            pltpu.SemaphoreType.DMA,
        ],
        compiler_params=pltpu.CompilerParams(needs_layout_passes=False),
    )


_hist_half = _make_hist(_N // 2)


_BINS_NP = np.arange(_NB, dtype=np.uint32)
_VMID_NP = ((_BINS_NP << _SHIFT) + (1 << (_SHIFT - 1))).view(np.float32)
_FIN_NP = np.isfinite(_VMID_NP)
_VMID = jnp.asarray(np.where(_FIN_NP, _VMID_NP, 0.0), dtype=jnp.float32)
_FIN = jnp.asarray(_FIN_NP.astype(np.float32))


def kernel(logits, targets):
    keys1, ps1 = _bce_keys_half(logits[:1], targets[:1])
    cnt_t1 = _hist_half(keys1.reshape(-1))
    keys2, ps2 = _bce_keys_half(logits[1:], targets[1:])
    cnt_t2 = _hist_half(keys2.reshape(-1))
    pos_sum = ps1 + ps2

    cnt = (jnp.sum(cnt_t1, axis=0) + jnp.sum(cnt_t2, axis=0)) * _FIN
    vsum = cnt * _VMID

    n_neg_f = jnp.sum(cnt)
    n_neg = n_neg_f.astype(jnp.int32)
    n_pos = _N - n_neg
    pos_mean = jnp.where(
        n_pos > 0,
        pos_sum[0, 0] / jnp.maximum(n_pos, 1).astype(jnp.float32),
        jnp.float32(0.0),
    )

    k = jnp.maximum(jnp.int32(1024), jnp.round(0.1 * n_neg_f).astype(jnp.int32))
    k = jnp.minimum(k, n_neg)
    k_f = k.astype(jnp.float32)

    csum = jnp.cumsum(cnt)
    above = n_neg_f - csum
    ge = above + cnt
    gt_mask = ge < k_f
    sel = (above < k_f) & (ge >= k_f)
    base_cnt = jnp.sum(jnp.where(gt_mask, cnt, 0.0))
    base_sum = jnp.sum(jnp.where(gt_mask, vsum, 0.0))
    vthr = jnp.sum(jnp.where(sel, _VMID, 0.0))
    hard_sum = base_sum + (k_f - base_cnt) * vthr
    neg_mean = jnp.where(
        n_neg > 0,
        hard_sum / jnp.maximum(k, 1).astype(jnp.float32),
        jnp.float32(0.0),
    )
    return pos_mean + neg_mean


# speedup vs baseline: 1.2801x; 1.2801x over previous
import jax
import jax.numpy as jnp
import numpy as np
from jax import lax
from jax.experimental import pallas as pl
from jax.experimental.pallas import tpu as pltpu
from jax.experimental.pallas import tpu_sc as plsc

_N = 2 * 1 * 128 * 128 * 128
_H = 2048
_W = 2048
_BH = 256

_NB = 65536
_SHIFT = 15
_NC = 2
_NS = 16
_NTILES = _NC * _NS
_PER_TILE = _N // _NTILES
_CHUNK = 8192
_NCHUNK = _PER_TILE // _CHUNK
_L = 16
_U = 16


_SHAPE = (2, 1, 128, 128, 128)
_BD = 32


def _bce_keys_body(l_ref, t_ref, key_ref, possum_ref):
    l = l_ref[...]
    t = t_ref[...]
    loss = jnp.maximum(l, 0.0) - l * t + jnp.log1p(jnp.exp(-jnp.abs(l)))
    neg = t <= 0.5
    digit_neg = lax.shift_right_logical(
        lax.bitcast_convert_type(loss, jnp.int32), _SHIFT)
    lane = lax.broadcasted_iota(jnp.int32, l.shape, dimension=4)
    digit_pos = jnp.int32(65280) + jnp.bitwise_and(lane, jnp.int32(127))
    key_ref[...] = jnp.where(neg, digit_neg, digit_pos)
    ps = jnp.sum(jnp.where(neg, 0.0, loss))

    @pl.when((pl.program_id(0) == 0) & (pl.program_id(1) == 0))
    def _():
        possum_ref[...] = jnp.zeros_like(possum_ref)

    possum_ref[...] += ps.reshape(1, 1)


def _make_bce(d0):
    return pl.pallas_call(
        _bce_keys_body,
        grid=(d0, 128 // _BD),
        in_specs=[
            pl.BlockSpec((1, 1, _BD, 128, 128), lambda i, j: (i, 0, j, 0, 0)),
            pl.BlockSpec((1, 1, _BD, 128, 128), lambda i, j: (i, 0, j, 0, 0)),
        ],
        out_specs=[
            pl.BlockSpec((1, 1, _BD, 128, 128), lambda i, j: (i, 0, j, 0, 0)),
            pl.BlockSpec((1, 1), lambda i, j: (0, 0)),
        ],
        out_shape=[
            jax.ShapeDtypeStruct((d0, 1, 128, 128, 128), jnp.int32),
            jax.ShapeDtypeStruct((1, 1), jnp.float32),
        ],
        compiler_params=pltpu.CompilerParams(
            dimension_semantics=("arbitrary", "arbitrary"),
        ),
    )


_bce_keys_full = _make_bce(2)


def _make_hist(n_elems):
    per_tile = n_elems // _NTILES
    nchunk = per_tile // _CHUNK

    def _hist_body(keys_hbm, cnt_out, chunk0, chunk1, cnt_v, sem0, sem1):
        wid = lax.axis_index("s") * _NC + lax.axis_index("c")
        zeros = jnp.zeros((_L,), jnp.float32)
        ones = jnp.ones((_L,), jnp.float32)

        def zero_body(i, carry):
            b = i * (_L * _U)
            for u in range(_U):
                cnt_v[pl.ds(b + u * _L, _L)] = zeros
            return carry

        lax.fori_loop(0, _NB // (_L * _U), zero_body, 0)

        base = wid * per_tile
        bufs = (chunk0, chunk1)
        sems = (sem0, sem1)

        def process(chunk_v):
            def vec_body(g, carry):
                b = g * (_L * _U)
                digits = [chunk_v[pl.ds(b + u * _L, _L)] for u in range(_U)]
                for digit in digits:
                    plsc.addupdate_scatter(cnt_v, [digit], ones)
                return carry

            lax.fori_loop(0, _CHUNK // (_L * _U), vec_body, 0)

        handles = [None] * nchunk
        handles[0] = pltpu.async_copy(
            keys_hbm.at[pl.ds(base, _CHUNK)], bufs[0], sems[0])
        for c in range(nchunk):
            if c + 1 < nchunk:
                handles[c + 1] = pltpu.async_copy(
                    keys_hbm.at[pl.ds(base + (c + 1) * _CHUNK, _CHUNK)],
                    bufs[(c + 1) % 2], sems[(c + 1) % 2])
            handles[c].wait()
            process(bufs[c % 2])

        pltpu.sync_copy(cnt_v, cnt_out.at[wid])

    return pl.kernel(
        _hist_body,
        out_type=jax.ShapeDtypeStruct((_NTILES, _NB), jnp.float32),
        mesh=plsc.VectorSubcoreMesh(core_axis_name="c", subcore_axis_name="s"),
        scratch_types=[
            pltpu.VMEM((_CHUNK,), jnp.int32),
            pltpu.VMEM((_CHUNK,), jnp.int32),
            pltpu.VMEM((_NB,), jnp.float32),
            pltpu.SemaphoreType.DMA,
            pltpu.SemaphoreType.DMA,
        ],
        compiler_params=pltpu.CompilerParams(needs_layout_passes=False),
    )


_hist_full = _make_hist(_N)


_BINS_NP = np.arange(_NB, dtype=np.uint32)
_VMID_NP = ((_BINS_NP << _SHIFT) + (1 << (_SHIFT - 1))).view(np.float32)
_FIN_NP = np.isfinite(_VMID_NP)
_VMID = jnp.asarray(np.where(_FIN_NP, _VMID_NP, 0.0), dtype=jnp.float32)
_FIN = jnp.asarray(_FIN_NP.astype(np.float32))


def kernel(logits, targets):
    keys, pos_sum = _bce_keys_full(logits, targets)
    cnt_tiles = _hist_full(keys.reshape(-1))

    cnt = jnp.sum(cnt_tiles, axis=0) * _FIN
    vsum = cnt * _VMID

    n_neg_f = jnp.sum(cnt)
    n_neg = n_neg_f.astype(jnp.int32)
    n_pos = _N - n_neg
    pos_mean = jnp.where(
        n_pos > 0,
        pos_sum[0, 0] / jnp.maximum(n_pos, 1).astype(jnp.float32),
        jnp.float32(0.0),
    )

    k = jnp.maximum(jnp.int32(1024), jnp.round(0.1 * n_neg_f).astype(jnp.int32))
    k = jnp.minimum(k, n_neg)
    k_f = k.astype(jnp.float32)

    csum = jnp.cumsum(cnt)
    above = n_neg_f - csum
    ge = above + cnt
    gt_mask = ge < k_f
    sel = (above < k_f) & (ge >= k_f)
    base_cnt = jnp.sum(jnp.where(gt_mask, cnt, 0.0))
    base_sum = jnp.sum(jnp.where(gt_mask, vsum, 0.0))
    vthr = jnp.sum(jnp.where(sel, _VMID, 0.0))
    hard_sum = base_sum + (k_f - base_cnt) * vthr
    neg_mean = jnp.where(
        n_neg > 0,
        hard_sum / jnp.maximum(k, 1).astype(jnp.float32),
        jnp.float32(0.0),
    )
    return pos_mean + neg_mean


# speedup vs baseline: 1.2890x; 1.0070x over previous
import jax
import jax.numpy as jnp
import numpy as np
from jax import lax
from jax.experimental import pallas as pl
from jax.experimental.pallas import tpu as pltpu
from jax.experimental.pallas import tpu_sc as plsc

_N = 2 * 1 * 128 * 128 * 128
_H = 2048
_W = 2048
_BH = 256

_NB = 65536
_SHIFT = 15
_NC = 2
_NS = 16
_NTILES = _NC * _NS
_PER_TILE = _N // _NTILES
_CHUNK = 8192
_NCHUNK = _PER_TILE // _CHUNK
_L = 16
_U = 8


_SHAPE = (2, 1, 128, 128, 128)
_BD = 32


def _bce_keys_body(l_ref, t_ref, key_ref, possum_ref):
    l = l_ref[...]
    t = t_ref[...]
    loss = jnp.maximum(l, 0.0) - l * t + jnp.log1p(jnp.exp(-jnp.abs(l)))
    neg = t <= 0.5
    digit_neg = lax.shift_right_logical(
        lax.bitcast_convert_type(loss, jnp.int32), _SHIFT)
    lane = lax.broadcasted_iota(jnp.int32, l.shape, dimension=4)
    digit_pos = jnp.int32(65280) + jnp.bitwise_and(lane, jnp.int32(127))
    key_ref[...] = jnp.where(neg, digit_neg, digit_pos)
    ps = jnp.sum(jnp.where(neg, 0.0, loss))

    @pl.when((pl.program_id(0) == 0) & (pl.program_id(1) == 0))
    def _():
        possum_ref[...] = jnp.zeros_like(possum_ref)

    possum_ref[...] += ps.reshape(1, 1)


def _make_bce(d0):
    return pl.pallas_call(
        _bce_keys_body,
        grid=(d0, 128 // _BD),
        in_specs=[
            pl.BlockSpec((1, 1, _BD, 128, 128), lambda i, j: (i, 0, j, 0, 0)),
            pl.BlockSpec((1, 1, _BD, 128, 128), lambda i, j: (i, 0, j, 0, 0)),
        ],
        out_specs=[
            pl.BlockSpec((1, 1, _BD, 128, 128), lambda i, j: (i, 0, j, 0, 0)),
            pl.BlockSpec((1, 1), lambda i, j: (0, 0)),
        ],
        out_shape=[
            jax.ShapeDtypeStruct((d0, 1, 128, 128, 128), jnp.int32),
            jax.ShapeDtypeStruct((1, 1), jnp.float32),
        ],
        compiler_params=pltpu.CompilerParams(
            dimension_semantics=("arbitrary", "arbitrary"),
        ),
    )


_bce_keys_full = _make_bce(2)


def _make_hist(n_elems):
    per_tile = n_elems // _NTILES
    nchunk = per_tile // _CHUNK

    def _hist_body(keys_hbm, cnt_out, chunk0, chunk1, cnt_v, sem0, sem1):
        wid = lax.axis_index("s") * _NC + lax.axis_index("c")
        zeros = jnp.zeros((_L,), jnp.float32)
        ones = jnp.ones((_L,), jnp.float32)

        def zero_body(i, carry):
            b = i * (_L * _U)
            for u in range(_U):
                cnt_v[pl.ds(b + u * _L, _L)] = zeros
            return carry

        lax.fori_loop(0, _NB // (_L * _U), zero_body, 0)

        base = wid * per_tile
        bufs = (chunk0, chunk1)
        sems = (sem0, sem1)

        def process(chunk_v):
            def vec_body(g, carry):
                b = g * (_L * _U)
                digits = [chunk_v[pl.ds(b + u * _L, _L)] for u in range(_U)]
                for digit in digits:
                    plsc.addupdate_scatter(cnt_v, [digit], ones)
                return carry

            lax.fori_loop(0, _CHUNK // (_L * _U), vec_body, 0)

        handles = [None] * nchunk
        handles[0] = pltpu.async_copy(
            keys_hbm.at[pl.ds(base, _CHUNK)], bufs[0], sems[0])
        for c in range(nchunk):
            if c + 1 < nchunk:
                handles[c + 1] = pltpu.async_copy(
                    keys_hbm.at[pl.ds(base + (c + 1) * _CHUNK, _CHUNK)],
                    bufs[(c + 1) % 2], sems[(c + 1) % 2])
            handles[c].wait()
            process(bufs[c % 2])

        pltpu.sync_copy(cnt_v, cnt_out.at[wid])

    return pl.kernel(
        _hist_body,
        out_type=jax.ShapeDtypeStruct((_NTILES, _NB), jnp.float32),
        mesh=plsc.VectorSubcoreMesh(core_axis_name="c", subcore_axis_name="s"),
        scratch_types=[
            pltpu.VMEM((_CHUNK,), jnp.int32),
            pltpu.VMEM((_CHUNK,), jnp.int32),
            pltpu.VMEM((_NB,), jnp.float32),
            pltpu.SemaphoreType.DMA,
            pltpu.SemaphoreType.DMA,
        ],
        compiler_params=pltpu.CompilerParams(needs_layout_passes=False),
    )


_hist_full = _make_hist(_N)


_BINS_NP = np.arange(_NB, dtype=np.uint32)
_VMID_NP = ((_BINS_NP << _SHIFT) + (1 << (_SHIFT - 1))).view(np.float32)
_FIN_NP = np.isfinite(_VMID_NP)
_VMID = jnp.asarray(np.where(_FIN_NP, _VMID_NP, 0.0), dtype=jnp.float32)
_FIN = jnp.asarray(_FIN_NP.astype(np.float32))


def kernel(logits, targets):
    keys, pos_sum = _bce_keys_full(logits, targets)
    cnt_tiles = _hist_full(keys.reshape(-1))

    cnt = jnp.sum(cnt_tiles, axis=0) * _FIN
    vsum = cnt * _VMID

    n_neg_f = jnp.sum(cnt)
    n_neg = n_neg_f.astype(jnp.int32)
    n_pos = _N - n_neg
    pos_mean = jnp.where(
        n_pos > 0,
        pos_sum[0, 0] / jnp.maximum(n_pos, 1).astype(jnp.float32),
        jnp.float32(0.0),
    )

    k = jnp.maximum(jnp.int32(1024), jnp.round(0.1 * n_neg_f).astype(jnp.int32))
    k = jnp.minimum(k, n_neg)
    k_f = k.astype(jnp.float32)

    csum = jnp.cumsum(cnt)
    above = n_neg_f - csum
    ge = above + cnt
    gt_mask = ge < k_f
    sel = (above < k_f) & (ge >= k_f)
    base_cnt = jnp.sum(jnp.where(gt_mask, cnt, 0.0))
    base_sum = jnp.sum(jnp.where(gt_mask, vsum, 0.0))
    vthr = jnp.sum(jnp.where(sel, _VMID, 0.0))
    hard_sum = base_sum + (k_f - base_cnt) * vthr
    neg_mean = jnp.where(
        n_neg > 0,
        hard_sum / jnp.maximum(k, 1).astype(jnp.float32),
        jnp.float32(0.0),
    )
    return pos_mean + neg_mean


# speedup vs baseline: 1.2913x; 1.0017x over previous
import jax
import jax.numpy as jnp
import numpy as np
from jax import lax
from jax.experimental import pallas as pl
from jax.experimental.pallas import tpu as pltpu
from jax.experimental.pallas import tpu_sc as plsc

_N = 2 * 1 * 128 * 128 * 128

_NB = 65536
_SHIFT = 15
_NC = 2
_NS = 16
_NTILES = _NC * _NS
_PER_TILE = _N // _NTILES
_CHUNK = 8192
_NCHUNK = _PER_TILE // _CHUNK
_L = 16
_U = 8


_BD = 32


def _bce_keys_body(l_ref, t_ref, key_ref, possum_ref):
    l = l_ref[...]
    t = t_ref[...]
    loss = jnp.maximum(l, 0.0) - l * t + jnp.log1p(jnp.exp(-jnp.abs(l)))
    neg = t <= 0.5
    digit_neg = lax.shift_right_logical(
        lax.bitcast_convert_type(loss, jnp.int32), _SHIFT)
    lane = lax.broadcasted_iota(jnp.int32, l.shape, dimension=4)
    digit_pos = jnp.int32(65280) + jnp.bitwise_and(lane, jnp.int32(127))
    key_ref[...] = jnp.where(neg, digit_neg, digit_pos)
    ps = jnp.sum(jnp.where(neg, 0.0, loss))

    @pl.when((pl.program_id(0) == 0) & (pl.program_id(1) == 0))
    def _():
        possum_ref[...] = jnp.zeros_like(possum_ref)

    possum_ref[...] += ps.reshape(1, 1)


def _make_bce(d0):
    return pl.pallas_call(
        _bce_keys_body,
        grid=(d0, 128 // _BD),
        in_specs=[
            pl.BlockSpec((1, 1, _BD, 128, 128), lambda i, j: (i, 0, j, 0, 0)),
            pl.BlockSpec((1, 1, _BD, 128, 128), lambda i, j: (i, 0, j, 0, 0)),
        ],
        out_specs=[
            pl.BlockSpec((1, 1, _BD, 128, 128), lambda i, j: (i, 0, j, 0, 0)),
            pl.BlockSpec((1, 1), lambda i, j: (0, 0)),
        ],
        out_shape=[
            jax.ShapeDtypeStruct((d0, 1, 128, 128, 128), jnp.int32),
            jax.ShapeDtypeStruct((1, 1), jnp.float32),
        ],
        compiler_params=pltpu.CompilerParams(
            dimension_semantics=("arbitrary", "arbitrary"),
        ),
    )


_bce_keys_full = _make_bce(2)


def _make_hist(n_elems):
    per_tile = n_elems // _NTILES
    nchunk = per_tile // _CHUNK

    def _hist_body(keys_hbm, cnt_out, chunk0, chunk1, cnt_v, sem0, sem1):
        wid = lax.axis_index("s") * _NC + lax.axis_index("c")
        zeros = jnp.zeros((_L,), jnp.float32)
        ones = jnp.ones((_L,), jnp.float32)

        def zero_body(i, carry):
            b = i * (_L * _U)
            for u in range(_U):
                cnt_v[pl.ds(b + u * _L, _L)] = zeros
            return carry

        lax.fori_loop(0, _NB // (_L * _U), zero_body, 0)

        base = wid * per_tile
        bufs = (chunk0, chunk1)
        sems = (sem0, sem1)

        def process(chunk_v):
            def vec_body(g, carry):
                b = g * (_L * _U)
                digits = [chunk_v[pl.ds(b + u * _L, _L)] for u in range(_U)]
                for digit in digits:
                    plsc.addupdate_scatter(cnt_v, [digit], ones)
                return carry

            lax.fori_loop(0, _CHUNK // (_L * _U), vec_body, 0)

        handles = [None] * nchunk
        handles[0] = pltpu.async_copy(
            keys_hbm.at[pl.ds(base, _CHUNK)], bufs[0], sems[0])
        for c in range(nchunk):
            if c + 1 < nchunk:
                handles[c + 1] = pltpu.async_copy(
                    keys_hbm.at[pl.ds(base + (c + 1) * _CHUNK, _CHUNK)],
                    bufs[(c + 1) % 2], sems[(c + 1) % 2])
            handles[c].wait()
            process(bufs[c % 2])

        pltpu.sync_copy(cnt_v, cnt_out.at[wid])

    return pl.kernel(
        _hist_body,
        out_type=jax.ShapeDtypeStruct((_NTILES, _NB), jnp.float32),
        mesh=plsc.VectorSubcoreMesh(core_axis_name="c", subcore_axis_name="s"),
        scratch_types=[
            pltpu.VMEM((_CHUNK,), jnp.int32),
            pltpu.VMEM((_CHUNK,), jnp.int32),
            pltpu.VMEM((_NB,), jnp.float32),
            pltpu.SemaphoreType.DMA,
            pltpu.SemaphoreType.DMA,
        ],
        compiler_params=pltpu.CompilerParams(needs_layout_passes=False),
    )


_hist_full = _make_hist(_N)


_BINS_NP = np.arange(_NB, dtype=np.uint32)
_VMID_NP = ((_BINS_NP << _SHIFT) + (1 << (_SHIFT - 1))).view(np.float32)
_FIN_NP = np.isfinite(_VMID_NP)
_VMID = jnp.asarray(np.where(_FIN_NP, _VMID_NP, 0.0), dtype=jnp.float32)
_FIN = jnp.asarray(_FIN_NP.astype(np.float32))


def kernel(logits, targets):
    keys, pos_sum = _bce_keys_full(logits, targets)
    cnt_tiles = _hist_full(keys.reshape(-1))

    cnt = jnp.sum(cnt_tiles, axis=0) * _FIN
    vsum = cnt * _VMID

    n_neg_f = jnp.sum(cnt)
    n_neg = n_neg_f.astype(jnp.int32)
    n_pos = _N - n_neg
    pos_mean = jnp.where(
        n_pos > 0,
        pos_sum[0, 0] / jnp.maximum(n_pos, 1).astype(jnp.float32),
        jnp.float32(0.0),
    )

    k = jnp.maximum(jnp.int32(1024), jnp.round(0.1 * n_neg_f).astype(jnp.int32))
    k = jnp.minimum(k, n_neg)
    k_f = k.astype(jnp.float32)

    csum = jnp.cumsum(cnt)
    above = n_neg_f - csum
    ge = above + cnt
    gt_mask = ge < k_f
    sel = (above < k_f) & (ge >= k_f)
    base_cnt = jnp.sum(jnp.where(gt_mask, cnt, 0.0))
    base_sum = jnp.sum(jnp.where(gt_mask, vsum, 0.0))
    vthr = jnp.sum(jnp.where(sel, _VMID, 0.0))
    hard_sum = base_sum + (k_f - base_cnt) * vthr
    neg_mean = jnp.where(
        n_neg > 0,
        hard_sum / jnp.maximum(k, 1).astype(jnp.float32),
        jnp.float32(0.0),
    )
    return pos_mean + neg_mean
